# Initial kernel scaffold; baseline (speedup 1.0000x reference)
import jax
import jax.numpy as jnp
from jax.experimental import pallas as pl


def kernel(x, edge_index, batch, W1, b1, W2, b2, W3, b3, Wl, bl):
    raise NotImplementedError("write your pallas kernel here")


# speedup vs baseline: 56.7987x; 56.7987x over previous
import functools

import jax
import jax.numpy as jnp
from jax import lax
from jax.experimental import pallas as pl
from jax.experimental.pallas import tpu as pltpu
from jax.experimental.pallas import tpu_sc as plsc

N = 100000
E = 6400000
G = 128
NC, NS = 2, 16
NW = NC * NS
LANE = 128
KB = 8

Np = 102400
ZR = Np // NS
EU = NW * LANE * KB
Ep = ((E + EU - 1) // EU) * EU
EROWS = Ep // LANE
ROWS_PW = EROWS // NW
ITERS = ROWS_PW // KB
BROWS = Np // LANE
BROWS_PW = BROWS // NW

_mesh = plsc.VectorSubcoreMesh(core_axis_name="c", subcore_axis_name="s")


def _make_agg(F):
  @functools.partial(
      pl.kernel,
      out_type=jax.ShapeDtypeStruct((NC, Np, F), jnp.float32),
      mesh=_mesh,
      scratch_types=[
          pltpu.VMEM((KB, LANE), jnp.int32),
          pltpu.VMEM((KB, LANE), jnp.int32),
          pltpu.VMEM((KB, LANE, F), jnp.float32),
          pltpu.VMEM_SHARED((Np, F), jnp.float32),
          pltpu.SemaphoreType.DMA,
          pltpu.SemaphoreType.DMA,
      ],
      compiler_params=pltpu.CompilerParams(
          needs_layout_passes=False, use_tc_tiling_on_sc=False),
  )
  def agg(z_hbm, src_hbm, dst_hbm, zeros_hbm, out_hbm,
          src_v, dst_v, rows_v, acc_sh, gsem, ssem):
    cid = lax.axis_index("c")
    sid = lax.axis_index("s")
    wid = cid * NS + sid
    pltpu.sync_copy(zeros_hbm.at[pl.ds(sid * ZR, ZR)],
                    acc_sh.at[pl.ds(sid * ZR, ZR)])
    plsc.subcore_barrier()
    row0 = wid * ROWS_PW

    @pl.loop(0, ITERS)
    def _(it):
      base = row0 + it * KB
      pltpu.sync_copy(src_hbm.at[pl.ds(base, KB)], src_v)
      pltpu.sync_copy(dst_hbm.at[pl.ds(base, KB)], dst_v)
      gds = [pltpu.async_copy(z_hbm.at[src_v.at[j]], rows_v.at[j], gsem)
             for j in range(KB)]
      for d in gds:
        d.wait()
      sds = [pltpu.async_copy(rows_v.at[j], acc_sh.at[dst_v.at[j]], ssem,
                              add=True)
             for j in range(KB)]
      for d in sds:
        d.wait()

    plsc.subcore_barrier()
    pltpu.sync_copy(acc_sh.at[pl.ds(sid * ZR, ZR)],
                    out_hbm.at[cid].at[pl.ds(sid * ZR, ZR)])

  return agg


def _make_counts(bins, rows_pw, iters, kb):
  @functools.partial(
      pl.kernel,
      out_type=jax.ShapeDtypeStruct((NW, bins), jnp.float32),
      mesh=_mesh,
      scratch_types=[
          pltpu.VMEM((kb, LANE), jnp.int32),
          pltpu.VMEM((bins,), jnp.float32),
      ],
      compiler_params=pltpu.CompilerParams(
          needs_layout_passes=False, use_tc_tiling_on_sc=False),
  )
  def cnt(idx_hbm, zeros_hbm, out_hbm, idx_v, acc_v):
    cid = lax.axis_index("c")
    sid = lax.axis_index("s")
    wid = cid * NS + sid
    pltpu.sync_copy(zeros_hbm, acc_v)
    ones = jnp.full((16,), 1.0, jnp.float32)
    row0 = wid * rows_pw

    @pl.loop(0, iters)
    def _(it):
      pltpu.sync_copy(idx_hbm.at[pl.ds(row0 + it * kb, kb)], idx_v)
      for j in range(kb):
        for c in range(LANE // 16):
          ids = idx_v[j, pl.ds(c * 16, 16)]
          plsc.addupdate_scatter(acc_v, [ids], ones)

    pltpu.sync_copy(acc_v, out_hbm.at[wid])

  return cnt


def _make_pool():
  @functools.partial(
      pl.kernel,
      out_type=jax.ShapeDtypeStruct((NC, 256, 16), jnp.float32),
      mesh=_mesh,
      scratch_types=[
          pltpu.VMEM((1, LANE), jnp.int32),
          pltpu.VMEM((LANE, 16), jnp.float32),
          pltpu.VMEM_SHARED((256, 16), jnp.float32),
          pltpu.SemaphoreType.DMA,
      ],
      compiler_params=pltpu.CompilerParams(
          needs_layout_passes=False, use_tc_tiling_on_sc=False),
  )
  def pool(m_hbm, bidx_hbm, zeros_hbm, out_hbm, bidx_v, val_v, acc_sh, ssem):
    cid = lax.axis_index("c")
    sid = lax.axis_index("s")
    wid = cid * NS + sid
    pltpu.sync_copy(zeros_hbm.at[pl.ds(sid * 16, 16)],
                    acc_sh.at[pl.ds(sid * 16, 16)])
    plsc.subcore_barrier()

    @pl.loop(0, BROWS_PW)
    def _(it):
      r = wid * BROWS_PW + it
      pltpu.sync_copy(bidx_hbm.at[pl.ds(r, 1)], bidx_v)
      pltpu.sync_copy(m_hbm.at[pl.ds(r * LANE, LANE)], val_v)
      pltpu.async_copy(val_v, acc_sh.at[bidx_v.at[0]], ssem, add=True).wait()

    plsc.subcore_barrier()

    @pl.when(sid == 0)
    def _():
      pltpu.sync_copy(acc_sh, out_hbm.at[cid])

  return pool


_agg8 = _make_agg(8)
_agg16 = _make_agg(16)
_cnt_deg = _make_counts(Np, ROWS_PW, ITERS, KB)
_cnt_batch = _make_counts(256, BROWS_PW, 5, 5)
_pool = _make_pool()


BR = 2048
TGRID = Np // BR


def _tc_prep(degp_t, x_p, W1):
  def body(degp_ref, x_ref, w_ref, dinv_ref, z1_ref):
    deg = jnp.sum(degp_ref[...], axis=1, keepdims=True) + 1.0
    dinv = lax.rsqrt(deg)
    dinv_ref[...] = dinv
    z1_ref[...] = jnp.dot(x_ref[...], w_ref[...],
                          preferred_element_type=jnp.float32) * dinv

  return pl.pallas_call(
      body,
      grid=(TGRID,),
      in_specs=[
          pl.BlockSpec((BR, NW), lambda i: (i, 0)),
          pl.BlockSpec((BR, 9), lambda i: (i, 0)),
          pl.BlockSpec((9, 8), lambda i: (0, 0)),
      ],
      out_specs=[
          pl.BlockSpec((BR, 1), lambda i: (i, 0)),
          pl.BlockSpec((BR, 8), lambda i: (i, 0)),
      ],
      out_shape=[
          jax.ShapeDtypeStruct((Np, 1), jnp.float32),
          jax.ShapeDtypeStruct((Np, 8), jnp.float32),
      ],
  )(degp_t, x_p, W1)


def _tc_layer(aggp, z, dinv, W, b, fin, fout, use_mm, use_relu, post_scale):
  def body(aggp_ref, z_ref, dinv_ref, w_ref, b_ref, out_ref):
    dv = dinv_ref[...]
    m = (aggp_ref[0] + aggp_ref[1] + z_ref[...]) * dv
    if use_mm:
      m = jnp.dot(m, w_ref[...], preferred_element_type=jnp.float32)
    if use_relu:
      m = jnp.maximum(m + b_ref[...], 0.0)
    if post_scale:
      m = m * dv
    out_ref[...] = m

  return pl.pallas_call(
      body,
      grid=(TGRID,),
      in_specs=[
          pl.BlockSpec((2, BR, fin), lambda i: (0, i, 0)),
          pl.BlockSpec((BR, fin), lambda i: (i, 0)),
          pl.BlockSpec((BR, 1), lambda i: (i, 0)),
          pl.BlockSpec((fin, fout), lambda i: (0, 0)),
          pl.BlockSpec((1, fout), lambda i: (0, 0)),
      ],
      out_specs=pl.BlockSpec((BR, fout), lambda i: (i, 0)),
      out_shape=jax.ShapeDtypeStruct((Np, fout), jnp.float32),
  )(aggp, z, dinv, W, b)


def _tc_head(poolp, cntp_t, W3, b3, Wl, bl):
  def body(pp_ref, cp_ref, w3_ref, b3_ref, wl_ref, bl_ref, out_ref):
    sums = (pp_ref[0] + pp_ref[1])[:G]
    cnts = jnp.sum(cp_ref[...], axis=1, keepdims=True)[:G]
    pooled = sums / jnp.maximum(cnts, 1.0)
    h = jnp.dot(pooled, w3_ref[...], preferred_element_type=jnp.float32)
    h = h + b3_ref[...]
    out = jnp.dot(h, wl_ref[...], preferred_element_type=jnp.float32)
    out_ref[...] = out + bl_ref[...]

  return pl.pallas_call(
      body,
      out_shape=jax.ShapeDtypeStruct((G, 2), jnp.float32),
  )(poolp, cntp_t, W3, b3, Wl, bl)


def kernel(x, edge_index, batch, W1, b1, W2, b2, W3, b3, Wl, bl):
  f32 = jnp.float32
  src = edge_index[0]
  dst = edge_index[1]
  epad = jnp.full((Ep - E,), N, jnp.int32)
  src_p = jnp.concatenate([src, epad]).reshape(EROWS, LANE)
  dst_p = jnp.concatenate([dst, epad]).reshape(EROWS, LANE)
  batch_p = jnp.concatenate(
      [batch, jnp.full((Np - N,), G, jnp.int32)]).reshape(BROWS, LANE)
  x_p = jnp.zeros((Np, 9), f32).at[:N].set(x)

  zeros_np = jnp.zeros((Np,), f32)
  zeros_256 = jnp.zeros((256,), f32)
  zeros_8 = jnp.zeros((Np, 8), f32)
  zeros_16 = jnp.zeros((Np, 16), f32)
  zeros_256x16 = jnp.zeros((256, 16), f32)

  degp = _cnt_deg(dst_p, zeros_np)
  cntp = _cnt_batch(batch_p, zeros_256)
  dinv, z1 = _tc_prep(degp.T, x_p, W1)

  agg1 = _agg8(z1, src_p, dst_p, zeros_8)
  z2 = _tc_layer(agg1, z1, dinv, jnp.zeros((8, 8), f32), b1.reshape(1, 8),
                 8, 8, use_mm=False, use_relu=True, post_scale=True)
  agg2 = _agg8(z2, src_p, dst_p, zeros_8)
  z3 = _tc_layer(agg2, z2, dinv, W2, b2.reshape(1, 16), 8, 16,
                 use_mm=True, use_relu=True, post_scale=True)
  agg3 = _agg16(z3, src_p, dst_p, zeros_16)
  m3 = _tc_layer(agg3, z3, dinv, jnp.zeros((16, 16), f32),
                 jnp.zeros((1, 16), f32), 16, 16,
                 use_mm=False, use_relu=False, post_scale=False)

  poolp = _pool(m3, batch_p, zeros_256x16)
  return _tc_head(poolp, cntp.T, W3, b3.reshape(1, 32), Wl, bl.reshape(1, 2))


# speedup vs baseline: 68.3005x; 1.2025x over previous
import functools

import jax
import jax.numpy as jnp
from jax import lax
from jax.experimental import pallas as pl
from jax.experimental.pallas import tpu as pltpu
from jax.experimental.pallas import tpu_sc as plsc

N = 100000
E = 6400000
G = 128
NC, NS = 2, 16
NW = NC * NS
LANE = 128

Np = 102400
ZR = Np // NS
EU = NW * LANE * 32
Ep = ((E + EU - 1) // EU) * EU
EROWS = Ep // LANE
ROWS_PW = EROWS // NW
CKB = 8
CITERS = ROWS_PW // CKB
BROWS = Np // LANE
BROWS_PW = BROWS // NW

_mesh = plsc.VectorSubcoreMesh(core_axis_name="c", subcore_axis_name="s")
_sc_params = pltpu.CompilerParams(
    needs_layout_passes=False, use_tc_tiling_on_sc=False)


def _make_agg(F, KB):
  ITERS = ROWS_PW // KB

  @functools.partial(
      pl.kernel,
      out_type=jax.ShapeDtypeStruct((NC, Np, F), jnp.float32),
      mesh=_mesh,
      scratch_types=[
          pltpu.VMEM((2, KB, 2, LANE), jnp.int32),
          pltpu.VMEM((2, KB, LANE, F), jnp.float32),
          pltpu.VMEM_SHARED((Np, F), jnp.float32),
          pltpu.SemaphoreType.DMA,
          pltpu.SemaphoreType.DMA,
      ],
      compiler_params=_sc_params,
  )
  def agg(z_hbm, idx_hbm, zeros_hbm, out_hbm,
          idx_v, rows_v, acc_sh, gsem, ssem):
    cid = lax.axis_index("c")
    sid = lax.axis_index("s")
    wid = cid * NS + sid
    pltpu.sync_copy(zeros_hbm.at[pl.ds(sid * ZR, ZR)],
                    acc_sh.at[pl.ds(sid * ZR, ZR)])
    plsc.subcore_barrier()
    row0 = wid * ROWS_PW

    def load_idx(b, it):
      pltpu.sync_copy(idx_hbm.at[pl.ds(row0 + it * KB, KB)], idx_v.at[b])

    def issue_gathers(b):
      return [pltpu.async_copy(z_hbm.at[idx_v.at[b, j, 0]],
                               rows_v.at[b].at[j], gsem)
              for j in range(KB)]

    def gathers_to_scatters(b, gds):
      for j in range(KB):
        gds[j].wait()
        pltpu.async_copy(rows_v.at[b].at[j], acc_sh.at[idx_v.at[b, j, 1]],
                         ssem, add=True)

    def drain_scatters(b):
      for j in range(KB):
        pltpu.make_async_copy(rows_v.at[b].at[j],
                              acc_sh.at[idx_v.at[b, j, 1]], ssem).wait()

    @pl.loop(0, ITERS // 2)
    def _(g):
      @pl.when(g > 0)
      def _():
        drain_scatters(0)
      load_idx(0, 2 * g)
      gds0 = issue_gathers(0)

      @pl.when(g > 0)
      def _():
        drain_scatters(1)
      gathers_to_scatters(0, gds0)
      load_idx(1, 2 * g + 1)
      gds1 = issue_gathers(1)
      gathers_to_scatters(1, gds1)

    drain_scatters(0)
    drain_scatters(1)
    plsc.subcore_barrier()
    pltpu.sync_copy(acc_sh.at[pl.ds(sid * ZR, ZR)],
                    out_hbm.at[cid].at[pl.ds(sid * ZR, ZR)])

  return agg


def _make_counts():
  @functools.partial(
      pl.kernel,
      out_type=(jax.ShapeDtypeStruct((NW, Np), jnp.float32),
                jax.ShapeDtypeStruct((NW, 256), jnp.float32)),
      mesh=_mesh,
      scratch_types=[
          pltpu.VMEM((2, CKB, LANE), jnp.int32),
          pltpu.VMEM((Np,), jnp.float32),
          pltpu.VMEM((256,), jnp.float32),
          pltpu.SemaphoreType.DMA,
      ],
      compiler_params=_sc_params,
  )
  def cnt(dst_hbm, b_hbm, zeros_hbm, zeros256_hbm, deg_hbm, bcnt_hbm,
          idx_v, acc_v, bacc_v, isem):
    cid = lax.axis_index("c")
    sid = lax.axis_index("s")
    wid = cid * NS + sid
    pltpu.sync_copy(zeros_hbm, acc_v)
    pltpu.sync_copy(zeros256_hbm, bacc_v)
    ones = jnp.full((16,), 1.0, jnp.float32)
    row0 = wid * ROWS_PW

    def issue_load(b, it):
      pltpu.async_copy(dst_hbm.at[pl.ds(row0 + it * CKB, CKB)],
                       idx_v.at[b], isem)

    def drain_load(b):
      pltpu.make_async_copy(dst_hbm.at[pl.ds(row0, CKB)],
                            idx_v.at[b], isem).wait()

    def process(b):
      for j in range(CKB):
        for c in range(LANE // 16):
          ids = idx_v[b, j, pl.ds(c * 16, 16)]
          plsc.addupdate_scatter(acc_v, [ids], ones)

    issue_load(0, 0)
    issue_load(1, 1)

    @pl.loop(0, CITERS // 2)
    def _(g):
      drain_load(0)
      process(0)

      @pl.when(2 * g + 2 < CITERS)
      def _():
        issue_load(0, 2 * g + 2)

      drain_load(1)
      process(1)

      @pl.when(2 * g + 3 < CITERS)
      def _():
        issue_load(1, 2 * g + 3)

    @pl.loop(0, BROWS_PW)
    def _(it):
      pltpu.sync_copy(b_hbm.at[pl.ds(wid * BROWS_PW + it, 1)],
                      idx_v.at[0, pl.ds(0, 1)])
      for c in range(LANE // 16):
        ids = idx_v[0, 0, pl.ds(c * 16, 16)]
        plsc.addupdate_scatter(bacc_v, [ids], ones)

    pltpu.sync_copy(acc_v, deg_hbm.at[wid])
    pltpu.sync_copy(bacc_v, bcnt_hbm.at[wid])

  return cnt


def _make_pool():
  @functools.partial(
      pl.kernel,
      out_type=jax.ShapeDtypeStruct((NC, 256, 16), jnp.float32),
      mesh=_mesh,
      scratch_types=[
          pltpu.VMEM((1, LANE), jnp.int32),
          pltpu.VMEM((LANE, 16), jnp.float32),
          pltpu.VMEM_SHARED((256, 16), jnp.float32),
          pltpu.SemaphoreType.DMA,
      ],
      compiler_params=_sc_params,
  )
  def pool(m_hbm, bidx_hbm, zeros_hbm, out_hbm, bidx_v, val_v, acc_sh, ssem):
    cid = lax.axis_index("c")
    sid = lax.axis_index("s")
    wid = cid * NS + sid
    pltpu.sync_copy(zeros_hbm.at[pl.ds(sid * 16, 16)],
                    acc_sh.at[pl.ds(sid * 16, 16)])
    plsc.subcore_barrier()

    @pl.loop(0, BROWS_PW)
    def _(it):
      r = wid * BROWS_PW + it
      pltpu.sync_copy(bidx_hbm.at[pl.ds(r, 1)], bidx_v)
      pltpu.sync_copy(m_hbm.at[pl.ds(r * LANE, LANE)], val_v)
      pltpu.async_copy(val_v, acc_sh.at[bidx_v.at[0]], ssem, add=True).wait()

    plsc.subcore_barrier()

    @pl.when(sid == 0)
    def _():
      pltpu.sync_copy(acc_sh, out_hbm.at[cid])

  return pool


_agg8 = _make_agg(8, 16)
_agg16 = _make_agg(16, 4)
_cnt_all = _make_counts()
_pool = _make_pool()


BR = 2048
TGRID = Np // BR


def _tc_prep(degp_t, x_p, W1):
  def body(degp_ref, x_ref, w_ref, dinv_ref, z1_ref):
    deg = jnp.sum(degp_ref[...], axis=1, keepdims=True) + 1.0
    dinv = lax.rsqrt(deg)
    dinv_ref[...] = dinv
    z1_ref[...] = jnp.dot(x_ref[...], w_ref[...],
                          preferred_element_type=jnp.float32) * dinv

  return pl.pallas_call(
      body,
      grid=(TGRID,),
      in_specs=[
          pl.BlockSpec((BR, NW), lambda i: (i, 0)),
          pl.BlockSpec((BR, 9), lambda i: (i, 0)),
          pl.BlockSpec((9, 8), lambda i: (0, 0)),
      ],
      out_specs=[
          pl.BlockSpec((BR, 1), lambda i: (i, 0)),
          pl.BlockSpec((BR, 8), lambda i: (i, 0)),
      ],
      out_shape=[
          jax.ShapeDtypeStruct((Np, 1), jnp.float32),
          jax.ShapeDtypeStruct((Np, 8), jnp.float32),
      ],
  )(degp_t, x_p, W1)


def _tc_layer(aggp, z, dinv, W, b, fin, fout, use_mm, use_relu, post_scale):
  def body(aggp_ref, z_ref, dinv_ref, w_ref, b_ref, out_ref):
    dv = dinv_ref[...]
    m = (aggp_ref[0] + aggp_ref[1] + z_ref[...]) * dv
    if use_mm:
      m = jnp.dot(m, w_ref[...], preferred_element_type=jnp.float32)
    if use_relu:
      m = jnp.maximum(m + b_ref[...], 0.0)
    if post_scale:
      m = m * dv
    out_ref[...] = m

  return pl.pallas_call(
      body,
      grid=(TGRID,),
      in_specs=[
          pl.BlockSpec((2, BR, fin), lambda i: (0, i, 0)),
          pl.BlockSpec((BR, fin), lambda i: (i, 0)),
          pl.BlockSpec((BR, 1), lambda i: (i, 0)),
          pl.BlockSpec((fin, fout), lambda i: (0, 0)),
          pl.BlockSpec((1, fout), lambda i: (0, 0)),
      ],
      out_specs=pl.BlockSpec((BR, fout), lambda i: (i, 0)),
      out_shape=jax.ShapeDtypeStruct((Np, fout), jnp.float32),
  )(aggp, z, dinv, W, b)


def _tc_head(poolp, cntp_t, W3, b3, Wl, bl):
  def body(pp_ref, cp_ref, w3_ref, b3_ref, wl_ref, bl_ref, out_ref):
    sums = (pp_ref[0] + pp_ref[1])[:G]
    cnts = jnp.sum(cp_ref[...], axis=1, keepdims=True)[:G]
    pooled = sums / jnp.maximum(cnts, 1.0)
    h = jnp.dot(pooled, w3_ref[...], preferred_element_type=jnp.float32)
    h = h + b3_ref[...]
    out = jnp.dot(h, wl_ref[...], preferred_element_type=jnp.float32)
    out_ref[...] = out + bl_ref[...]

  return pl.pallas_call(
      body,
      out_shape=jax.ShapeDtypeStruct((G, 2), jnp.float32),
  )(poolp, cntp_t, W3, b3, Wl, bl)


def kernel(x, edge_index, batch, W1, b1, W2, b2, W3, b3, Wl, bl):
  f32 = jnp.float32
  epad = jnp.full((Ep - E,), N, jnp.int32)
  src_p = jnp.concatenate([edge_index[0], epad]).reshape(EROWS, 1, LANE)
  dst_p = jnp.concatenate([edge_index[1], epad]).reshape(EROWS, 1, LANE)
  idx_p = jnp.concatenate([src_p, dst_p], axis=1)
  batch_p = jnp.concatenate(
      [batch, jnp.full((Np - N,), G, jnp.int32)]).reshape(BROWS, LANE)
  x_p = jnp.zeros((Np, 9), f32).at[:N].set(x)

  zeros_np = jnp.zeros((Np,), f32)
  zeros_256 = jnp.zeros((256,), f32)
  zeros_8 = jnp.zeros((Np, 8), f32)
  zeros_16 = jnp.zeros((Np, 16), f32)
  zeros_256x16 = jnp.zeros((256, 16), f32)

  degp, cntp = _cnt_all(dst_p.reshape(EROWS, LANE), batch_p,
                        zeros_np, zeros_256)
  dinv, z1 = _tc_prep(degp.T, x_p, W1)

  agg1 = _agg8(z1, idx_p, zeros_8)
  z2 = _tc_layer(agg1, z1, dinv, jnp.zeros((8, 8), f32), b1.reshape(1, 8),
                 8, 8, use_mm=False, use_relu=True, post_scale=True)
  agg2 = _agg8(z2, idx_p, zeros_8)
  z3 = _tc_layer(agg2, z2, dinv, W2, b2.reshape(1, 16), 8, 16,
                 use_mm=True, use_relu=True, post_scale=True)
  agg3 = _agg16(z3, idx_p, zeros_16)
  m3 = _tc_layer(agg3, z3, dinv, jnp.zeros((16, 16), f32),
                 jnp.zeros((1, 16), f32), 16, 16,
                 use_mm=False, use_relu=False, post_scale=False)

  poolp = _pool(m3, batch_p, zeros_256x16)
  return _tc_head(poolp, cntp.T, W3, b3.reshape(1, 32), Wl, bl.reshape(1, 2))


# speedup vs baseline: 75.0799x; 1.0993x over previous
import functools

import jax
import jax.numpy as jnp
from jax import lax
from jax.experimental import pallas as pl
from jax.experimental.pallas import tpu as pltpu
from jax.experimental.pallas import tpu_sc as plsc

N = 100000
E = 6400000
G = 128
NC, NS = 2, 16
NW = NC * NS
LANE = 128

Np = 102400
ZR = Np // NS
EU = NW * LANE * 32
Ep = ((E + EU - 1) // EU) * EU
EROWS = Ep // LANE
ROWS_PW = EROWS // NW
CKB = 16
CITERS = ROWS_PW // CKB
BROWS = Np // LANE
BROWS_PW = BROWS // NW

_mesh = plsc.VectorSubcoreMesh(core_axis_name="c", subcore_axis_name="s")
_sc_params = pltpu.CompilerParams(
    needs_layout_passes=False, use_tc_tiling_on_sc=False)


def _make_agg(F, KB, dtype):
  ITERS = ROWS_PW // KB

  @functools.partial(
      pl.kernel,
      out_type=jax.ShapeDtypeStruct((NC, Np, F), dtype),
      mesh=_mesh,
      scratch_types=[
          pltpu.VMEM((2, KB, LANE), jnp.int32),
          pltpu.VMEM((2, KB, LANE), jnp.int32),
          pltpu.VMEM((2, KB, LANE, F), dtype),
          pltpu.VMEM_SHARED((Np, F), dtype),
          pltpu.SemaphoreType.DMA,
          pltpu.SemaphoreType.DMA,
      ],
      compiler_params=_sc_params,
  )
  def agg(z_hbm, src_hbm, dst_hbm, zeros_hbm, out_hbm,
          src_v, dst_v, rows_v, acc_sh, gsem, ssem):
    cid = lax.axis_index("c")
    sid = lax.axis_index("s")
    wid = cid * NS + sid
    pltpu.sync_copy(zeros_hbm.at[pl.ds(sid * ZR, ZR)],
                    acc_sh.at[pl.ds(sid * ZR, ZR)])
    plsc.subcore_barrier()
    row0 = wid * ROWS_PW

    def load_idx(b, it):
      pltpu.sync_copy(src_hbm.at[pl.ds(row0 + it * KB, KB)], src_v.at[b])
      pltpu.sync_copy(dst_hbm.at[pl.ds(row0 + it * KB, KB)], dst_v.at[b])

    def issue_gathers(b):
      return [pltpu.async_copy(z_hbm.at[src_v.at[b, j]],
                               rows_v.at[b].at[j], gsem)
              for j in range(KB)]

    def gathers_to_scatters(b, gds):
      for j in range(KB):
        gds[j].wait()
        pltpu.async_copy(rows_v.at[b].at[j], acc_sh.at[dst_v.at[b, j]],
                         ssem, add=True)

    def drain_scatters(b):
      for j in range(KB):
        pltpu.make_async_copy(rows_v.at[b].at[j],
                              acc_sh.at[dst_v.at[b, j]], ssem).wait()

    @pl.loop(0, ITERS // 2)
    def _(g):
      @pl.when(g > 0)
      def _():
        drain_scatters(0)
      load_idx(0, 2 * g)
      gds0 = issue_gathers(0)

      @pl.when(g > 0)
      def _():
        drain_scatters(1)
      gathers_to_scatters(0, gds0)
      load_idx(1, 2 * g + 1)
      gds1 = issue_gathers(1)
      gathers_to_scatters(1, gds1)

    drain_scatters(0)
    drain_scatters(1)
    plsc.subcore_barrier()
    pltpu.sync_copy(acc_sh.at[pl.ds(sid * ZR, ZR)],
                    out_hbm.at[cid].at[pl.ds(sid * ZR, ZR)])

  return agg


def _make_counts():
  @functools.partial(
      pl.kernel,
      out_type=(jax.ShapeDtypeStruct((NW, Np), jnp.float32),
                jax.ShapeDtypeStruct((NW, 256), jnp.float32)),
      mesh=_mesh,
      scratch_types=[
          pltpu.VMEM((2, CKB, LANE), jnp.int32),
          pltpu.VMEM((Np,), jnp.float32),
          pltpu.VMEM((256,), jnp.float32),
          pltpu.SemaphoreType.DMA,
      ],
      compiler_params=_sc_params,
  )
  def cnt(dst_hbm, b_hbm, zeros_hbm, zeros256_hbm, deg_hbm, bcnt_hbm,
          idx_v, acc_v, bacc_v, isem):
    cid = lax.axis_index("c")
    sid = lax.axis_index("s")
    wid = cid * NS + sid
    pltpu.sync_copy(zeros_hbm, acc_v)
    pltpu.sync_copy(zeros256_hbm, bacc_v)
    ones = jnp.full((16,), 1.0, jnp.float32)
    row0 = wid * ROWS_PW

    def issue_load(b, it):
      pltpu.async_copy(dst_hbm.at[pl.ds(row0 + it * CKB, CKB)],
                       idx_v.at[b], isem)

    def drain_load(b):
      pltpu.make_async_copy(dst_hbm.at[pl.ds(row0, CKB)],
                            idx_v.at[b], isem).wait()

    def process(b):
      for j in range(CKB):
        for c in range(LANE // 16):
          ids = idx_v[b, j, pl.ds(c * 16, 16)]
          plsc.addupdate_scatter(acc_v, [ids], ones)

    issue_load(0, 0)
    issue_load(1, 1)

    @pl.loop(0, CITERS // 2)
    def _(g):
      drain_load(0)
      process(0)

      @pl.when(2 * g + 2 < CITERS)
      def _():
        issue_load(0, 2 * g + 2)

      drain_load(1)
      process(1)

      @pl.when(2 * g + 3 < CITERS)
      def _():
        issue_load(1, 2 * g + 3)

    @pl.loop(0, BROWS_PW)
    def _(it):
      pltpu.sync_copy(b_hbm.at[pl.ds(wid * BROWS_PW + it, 1)],
                      idx_v.at[0, pl.ds(0, 1)])
      for c in range(LANE // 16):
        ids = idx_v[0, 0, pl.ds(c * 16, 16)]
        plsc.addupdate_scatter(bacc_v, [ids], ones)

    pltpu.sync_copy(acc_v, deg_hbm.at[wid])
    pltpu.sync_copy(bacc_v, bcnt_hbm.at[wid])

  return cnt


def _make_pool():
  @functools.partial(
      pl.kernel,
      out_type=jax.ShapeDtypeStruct((NC, 256, 16), jnp.float32),
      mesh=_mesh,
      scratch_types=[
          pltpu.VMEM((1, LANE), jnp.int32),
          pltpu.VMEM((LANE, 16), jnp.float32),
          pltpu.VMEM_SHARED((256, 16), jnp.float32),
          pltpu.SemaphoreType.DMA,
      ],
      compiler_params=_sc_params,
  )
  def pool(m_hbm, bidx_hbm, zeros_hbm, out_hbm, bidx_v, val_v, acc_sh, ssem):
    cid = lax.axis_index("c")
    sid = lax.axis_index("s")
    wid = cid * NS + sid
    pltpu.sync_copy(zeros_hbm.at[pl.ds(sid * 16, 16)],
                    acc_sh.at[pl.ds(sid * 16, 16)])
    plsc.subcore_barrier()

    @pl.loop(0, BROWS_PW)
    def _(it):
      r = wid * BROWS_PW + it
      pltpu.sync_copy(bidx_hbm.at[pl.ds(r, 1)], bidx_v)
      pltpu.sync_copy(m_hbm.at[pl.ds(r * LANE, LANE)], val_v)
      pltpu.async_copy(val_v, acc_sh.at[bidx_v.at[0]], ssem, add=True).wait()

    plsc.subcore_barrier()

    @pl.when(sid == 0)
    def _():
      pltpu.sync_copy(acc_sh, out_hbm.at[cid])

  return pool


_agg8 = _make_agg(8, 16, jnp.float32)
_agg16 = _make_agg(16, 16, jnp.bfloat16)
_cnt_all = _make_counts()
_pool = _make_pool()


BR = 2048
TGRID = Np // BR


def _tc_prep(degp_t, x_p, W1):
  def body(degp_ref, x_ref, w_ref, dinv_ref, z1_ref):
    deg = jnp.sum(degp_ref[...], axis=1, keepdims=True) + 1.0
    dinv = lax.rsqrt(deg)
    dinv_ref[...] = dinv
    z1_ref[...] = jnp.dot(x_ref[...], w_ref[...],
                          preferred_element_type=jnp.float32) * dinv

  return pl.pallas_call(
      body,
      grid=(TGRID,),
      in_specs=[
          pl.BlockSpec((BR, NW), lambda i: (i, 0)),
          pl.BlockSpec((BR, 9), lambda i: (i, 0)),
          pl.BlockSpec((9, 8), lambda i: (0, 0)),
      ],
      out_specs=[
          pl.BlockSpec((BR, 1), lambda i: (i, 0)),
          pl.BlockSpec((BR, 8), lambda i: (i, 0)),
      ],
      out_shape=[
          jax.ShapeDtypeStruct((Np, 1), jnp.float32),
          jax.ShapeDtypeStruct((Np, 8), jnp.float32),
      ],
  )(degp_t, x_p, W1)


def _tc_layer(aggp, z, dinv, W, b, fin, fout, use_mm, use_relu, post_scale,
              out_dtype=jnp.float32):
  def body(aggp_ref, z_ref, dinv_ref, w_ref, b_ref, out_ref):
    dv = dinv_ref[...]
    a = (aggp_ref[0].astype(jnp.float32) + aggp_ref[1].astype(jnp.float32)
         + z_ref[...].astype(jnp.float32))
    m = a * dv
    if use_mm:
      m = jnp.dot(m, w_ref[...], preferred_element_type=jnp.float32)
    if use_relu:
      m = jnp.maximum(m + b_ref[...], 0.0)
    if post_scale:
      m = m * dv
    out_ref[...] = m.astype(out_dtype)

  return pl.pallas_call(
      body,
      grid=(TGRID,),
      in_specs=[
          pl.BlockSpec((2, BR, fin), lambda i: (0, i, 0)),
          pl.BlockSpec((BR, fin), lambda i: (i, 0)),
          pl.BlockSpec((BR, 1), lambda i: (i, 0)),
          pl.BlockSpec((fin, fout), lambda i: (0, 0)),
          pl.BlockSpec((1, fout), lambda i: (0, 0)),
      ],
      out_specs=pl.BlockSpec((BR, fout), lambda i: (i, 0)),
      out_shape=jax.ShapeDtypeStruct((Np, fout), out_dtype),
  )(aggp, z, dinv, W, b)


def _tc_head(poolp, cntp_t, W3, b3, Wl, bl):
  def body(pp_ref, cp_ref, w3_ref, b3_ref, wl_ref, bl_ref, out_ref):
    sums = (pp_ref[0] + pp_ref[1])[:G]
    cnts = jnp.sum(cp_ref[...], axis=1, keepdims=True)[:G]
    pooled = sums / jnp.maximum(cnts, 1.0)
    h = jnp.dot(pooled, w3_ref[...], preferred_element_type=jnp.float32)
    h = h + b3_ref[...]
    out = jnp.dot(h, wl_ref[...], preferred_element_type=jnp.float32)
    out_ref[...] = out + bl_ref[...]

  return pl.pallas_call(
      body,
      out_shape=jax.ShapeDtypeStruct((G, 2), jnp.float32),
  )(poolp, cntp_t, W3, b3, Wl, bl)


def kernel(x, edge_index, batch, W1, b1, W2, b2, W3, b3, Wl, bl):
  f32 = jnp.float32
  bf16 = jnp.bfloat16
  epad = jnp.full((Ep - E,), N, jnp.int32)
  src_p = jnp.concatenate([edge_index[0], epad]).reshape(EROWS, LANE)
  dst_p = jnp.concatenate([edge_index[1], epad]).reshape(EROWS, LANE)
  batch_p = jnp.concatenate(
      [batch, jnp.full((Np - N,), G, jnp.int32)]).reshape(BROWS, LANE)
  x_p = jnp.zeros((Np, 9), f32).at[:N].set(x)

  zeros_np = jnp.zeros((Np,), f32)
  zeros_256 = jnp.zeros((256,), f32)
  zeros_8 = jnp.zeros((Np, 8), f32)
  zeros_16 = jnp.zeros((Np, 16), bf16)
  zeros_256x16 = jnp.zeros((256, 16), f32)

  degp, cntp = _cnt_all(dst_p, batch_p,
                        zeros_np, zeros_256)
  dinv, z1 = _tc_prep(degp.T, x_p, W1)

  agg1 = _agg8(z1, src_p, dst_p, zeros_8)
  z2 = _tc_layer(agg1, z1, dinv, jnp.zeros((8, 8), f32), b1.reshape(1, 8),
                 8, 8, use_mm=False, use_relu=True, post_scale=True)
  agg2 = _agg8(z2, src_p, dst_p, zeros_8)
  z3 = _tc_layer(agg2, z2, dinv, W2, b2.reshape(1, 16), 8, 16,
                 use_mm=True, use_relu=True, post_scale=True,
                 out_dtype=bf16)
  agg3 = _agg16(z3, src_p, dst_p, zeros_16)
  m3 = _tc_layer(agg3, z3, dinv, jnp.zeros((16, 16), f32),
                 jnp.zeros((1, 16), f32), 16, 16,
                 use_mm=False, use_relu=False, post_scale=False)

  poolp = _pool(m3, batch_p, zeros_256x16)
  return _tc_head(poolp, cntp.T, W3, b3.reshape(1, 32), Wl, bl.reshape(1, 2))
